# Initial kernel scaffold; baseline (speedup 1.0000x reference)
import jax
import jax.numpy as jnp
from jax.experimental import pallas as pl


def kernel(pitch, amplitude, wavetables, attention):
    raise NotImplementedError("write your pallas kernel here")


# speedup vs baseline: 46.1629x; 46.1629x over previous
import functools

import jax
import jax.numpy as jnp
from jax import lax
from jax.experimental import pallas as pl
from jax.experimental.pallas import tpu as pltpu
from jax.experimental.pallas import tpu_sc as plsc

_N_WT = 64
_L = 512
_SR = 16000
_B = 8
_T = 64000
_BLOCK = 160
_NBLK = _T // _BLOCK
_ROWS = 500
_LANES = 128

_NW = 32
_SPW = _B * _T // _NW
_BPW = _SPW // _BLOCK
_SUBBLK = 20
_SUBSAMP = _SUBBLK * _BLOCK
_NSTAGE = _BPW // _SUBBLK


def _prep_body(pitch_ref, wt_ref, att_ref, ph_ref, m_ref, d_ref):
    w = wt_ref[...]
    w = jnp.concatenate([w[:4], jnp.tanh(w[4:])], axis=0)
    a = att_ref[...]
    a = a - jnp.max(a, axis=0, keepdims=True)
    e = jnp.exp(a)
    att = e / jnp.sum(e, axis=0, keepdims=True)
    m = lax.dot_general(att, w, (((0,), (0,)), ((), ())),
                        preferred_element_type=jnp.float32)
    m_ref[...] = m
    d_ref[...] = jnp.concatenate([m[:, 1:], m[:, :1]], axis=1) - m

    inc = pitch_ref[...] / jnp.float32(_SR) * jnp.float32(_L)
    y = inc
    sh = 1
    while sh < _LANES:
        z = jnp.zeros((_B, _ROWS, sh), jnp.float32)
        y = y + jnp.concatenate([z, y[:, :, :_LANES - sh]], axis=2)
        sh *= 2
    rows = y[:, :, _LANES - 1]
    s = rows
    sh = 1
    while sh < _ROWS:
        z = jnp.zeros((_B, sh), jnp.float32)
        s = s + jnp.concatenate([z, s[:, :_ROWS - sh]], axis=1)
        s = s - jnp.float32(_L) * jnp.floor(s * jnp.float32(1.0 / _L))
        sh *= 2
    ex = jnp.concatenate([jnp.zeros((_B, 1), jnp.float32), s[:, :_ROWS - 1]],
                         axis=1)
    idx = y + ex[:, :, None] - inc[0:1]
    ph = idx - jnp.float32(_L) * jnp.floor(idx * jnp.float32(1.0 / _L))
    ph = jnp.where(ph >= jnp.float32(_L), ph - jnp.float32(_L), ph)
    ph_ref[...] = ph


def _sc_body(ph_hbm, amp_hbm, m_hbm, d_hbm, out_hbm,
             ph_v, amp_v, out_v, mt_v, dt_v):
    cid = lax.axis_index("c")
    sid = lax.axis_index("s")
    wid = sid * 2 + cid
    base = wid * _SPW
    blk_base = (wid % 4) * _BPW

    for st in range(_NSTAGE):
        samp0 = base + st * _SUBSAMP
        tab0 = (blk_base + st * _SUBBLK) * _L
        pltpu.sync_copy(ph_hbm.at[pl.ds(samp0, _SUBSAMP)], ph_v)
        pltpu.sync_copy(amp_hbm.at[pl.ds(samp0, _SUBSAMP)], amp_v)
        pltpu.sync_copy(m_hbm.at[pl.ds(tab0, _SUBBLK * _L)], mt_v)
        pltpu.sync_copy(d_hbm.at[pl.ds(tab0, _SUBBLK * _L)], dt_v)

        def blk_body(bl, carry):
            gbase = bl * _L
            for v in range(_BLOCK // 16):
                off = bl * _BLOCK + v * 16
                ph = ph_v[pl.ds(off, 16)]
                lo = ph.astype(jnp.int32)
                lo = jnp.minimum(jnp.maximum(lo, 0), _L - 1)
                alpha = ph - lo.astype(jnp.float32)
                g = lo + gbase
                mval = plsc.load_gather(mt_v, [g])
                dval = plsc.load_gather(dt_v, [g])
                amp = amp_v[pl.ds(off, 16)]
                out_v[pl.ds(off, 16)] = amp * (mval + alpha * dval)
            return carry

        lax.fori_loop(0, _SUBBLK, blk_body, 0)
        pltpu.sync_copy(out_v, out_hbm.at[pl.ds(samp0, _SUBSAMP)])


def kernel(pitch, amplitude, wavetables, attention):
    pitch3 = pitch.reshape(_B, _ROWS, _LANES)
    ph, m, d = pl.pallas_call(
        _prep_body,
        out_shape=(
            jax.ShapeDtypeStruct((_B, _ROWS, _LANES), jnp.float32),
            jax.ShapeDtypeStruct((_NBLK, _L), jnp.float32),
            jax.ShapeDtypeStruct((_NBLK, _L), jnp.float32),
        ),
    )(pitch3, wavetables, attention)

    mesh = plsc.VectorSubcoreMesh(core_axis_name="c", subcore_axis_name="s")
    sc = functools.partial(
        pl.kernel,
        mesh=mesh,
        compiler_params=pltpu.CompilerParams(needs_layout_passes=False),
        out_type=jax.ShapeDtypeStruct((_B * _T,), jnp.float32),
        scratch_types=[
            pltpu.VMEM((_SUBSAMP,), jnp.float32),
            pltpu.VMEM((_SUBSAMP,), jnp.float32),
            pltpu.VMEM((_SUBSAMP,), jnp.float32),
            pltpu.VMEM((_SUBBLK * _L,), jnp.float32),
            pltpu.VMEM((_SUBBLK * _L,), jnp.float32),
        ],
    )(_sc_body)
    out = sc(ph.reshape(_B * _T), amplitude.reshape(_B * _T),
             m.reshape(_NBLK * _L), d.reshape(_NBLK * _L))
    return out.reshape(_B, _T, 1)


# speedup vs baseline: 57.0155x; 1.2351x over previous
import functools

import jax
import jax.numpy as jnp
from jax import lax
from jax.experimental import pallas as pl
from jax.experimental.pallas import tpu as pltpu
from jax.experimental.pallas import tpu_sc as plsc

_N_WT = 64
_L = 512
_SR = 16000
_B = 8
_T = 64000
_BLOCK = 160
_NBLK = _T // _BLOCK
_ROWS = 500
_LANES = 128

_NW = 32
_SPW = _B * _T // _NW
_BPW = _SPW // _BLOCK
_SUBBLK = 20
_SUBSAMP = _SUBBLK * _BLOCK
_NSTAGE = _BPW // _SUBBLK


def _prep_body(pitch_ref, wt_ref, att_ref, ph_ref, m_ref, d_ref):
    w = wt_ref[...]
    w = jnp.concatenate([w[:4], jnp.tanh(w[4:])], axis=0)
    a = att_ref[...]
    a = a - jnp.max(a, axis=0, keepdims=True)
    e = jnp.exp(a)
    att = e / jnp.sum(e, axis=0, keepdims=True)
    m = lax.dot_general(att, w, (((0,), (0,)), ((), ())),
                        preferred_element_type=jnp.float32)
    m_ref[...] = m
    d_ref[...] = jnp.concatenate([m[:, 1:], m[:, :1]], axis=1) - m

    inc = pitch_ref[...] / jnp.float32(_SR) * jnp.float32(_L)
    y = inc
    sh = 1
    while sh < _LANES:
        z = jnp.zeros((_B, _ROWS, sh), jnp.float32)
        y = y + jnp.concatenate([z, y[:, :, :_LANES - sh]], axis=2)
        sh *= 2
    rows = y[:, :, _LANES - 1]
    s = rows
    sh = 1
    while sh < _ROWS:
        z = jnp.zeros((_B, sh), jnp.float32)
        s = s + jnp.concatenate([z, s[:, :_ROWS - sh]], axis=1)
        s = s - jnp.float32(_L) * jnp.floor(s * jnp.float32(1.0 / _L))
        sh *= 2
    ex = jnp.concatenate([jnp.zeros((_B, 1), jnp.float32), s[:, :_ROWS - 1]],
                         axis=1)
    idx = y + ex[:, :, None] - inc[0:1]
    ph = idx - jnp.float32(_L) * jnp.floor(idx * jnp.float32(1.0 / _L))
    ph = jnp.where(ph >= jnp.float32(_L), ph - jnp.float32(_L), ph)
    ph_ref[...] = ph


_WBLK = 13
_WSAMP = _WBLK * _BLOCK


def _sc_body(ph_hbm, amp_hbm, m_hbm, d_hbm, out_hbm,
             ph_v, amp_v, out_v, mt_v, dt_v, sem):
    cid = lax.axis_index("c")
    sid = lax.axis_index("s")
    wid = sid * 2 + cid
    blk0 = (wid * 25) // 2
    t0 = blk0 * _BLOCK

    cp = pltpu.make_async_copy
    dmas = [
        cp(m_hbm.at[pl.ds(blk0 * _L, _WBLK * _L)], mt_v, sem),
        cp(d_hbm.at[pl.ds(blk0 * _L, _WBLK * _L)], dt_v, sem),
    ]
    for b in range(_B):
        src = pl.ds(b * _T + t0, _WSAMP)
        dst = pl.ds(b * _WSAMP, _WSAMP)
        dmas.append(cp(ph_hbm.at[src], ph_v.at[dst], sem))
        dmas.append(cp(amp_hbm.at[src], amp_v.at[dst], sem))
    for dma in dmas:
        dma.start()
    for dma in dmas:
        dma.wait()

    for b in range(_B):
        def blk_body(bl, carry, b=b):
            gbase = bl * _L
            for v in range(_BLOCK // 16):
                off = b * _WSAMP + bl * _BLOCK + v * 16
                ph = ph_v[pl.ds(off, 16)]
                lo = ph.astype(jnp.int32)
                lo = jnp.minimum(jnp.maximum(lo, 0), _L - 1)
                alpha = ph - lo.astype(jnp.float32)
                g = lo + gbase
                mval = plsc.load_gather(mt_v, [g])
                dval = plsc.load_gather(dt_v, [g])
                amp = amp_v[pl.ds(off, 16)]
                out_v[pl.ds(off, 16)] = amp * (mval + alpha * dval)
            return carry

        lax.fori_loop(0, _WBLK, blk_body, 0)

    odmas = [cp(out_v.at[pl.ds(b * _WSAMP, _WSAMP)],
                out_hbm.at[pl.ds(b * _T + t0, _WSAMP)], sem)
             for b in range(_B)]
    for dma in odmas:
        dma.start()
    for dma in odmas:
        dma.wait()


def kernel(pitch, amplitude, wavetables, attention):
    pitch3 = pitch.reshape(_B, _ROWS, _LANES)
    ph, m, d = pl.pallas_call(
        _prep_body,
        out_shape=(
            jax.ShapeDtypeStruct((_B, _ROWS, _LANES), jnp.float32),
            jax.ShapeDtypeStruct((_NBLK, _L), jnp.float32),
            jax.ShapeDtypeStruct((_NBLK, _L), jnp.float32),
        ),
    )(pitch3, wavetables, attention)

    mesh = plsc.VectorSubcoreMesh(core_axis_name="c", subcore_axis_name="s")
    sc = functools.partial(
        pl.kernel,
        mesh=mesh,
        compiler_params=pltpu.CompilerParams(needs_layout_passes=False),
        out_type=jax.ShapeDtypeStruct((_B * _T,), jnp.float32),
        scratch_types=[
            pltpu.VMEM((_B * _WSAMP,), jnp.float32),
            pltpu.VMEM((_B * _WSAMP,), jnp.float32),
            pltpu.VMEM((_B * _WSAMP,), jnp.float32),
            pltpu.VMEM((_WBLK * _L,), jnp.float32),
            pltpu.VMEM((_WBLK * _L,), jnp.float32),
            pltpu.SemaphoreType.DMA,
        ],
    )(_sc_body)
    out = sc(ph.reshape(_B * _T), amplitude.reshape(_B * _T),
             m.reshape(_NBLK * _L), d.reshape(_NBLK * _L))
    return out.reshape(_B, _T, 1)


# speedup vs baseline: 77.8545x; 1.3655x over previous
import functools

import jax
import jax.numpy as jnp
from jax import lax
from jax.experimental import pallas as pl
from jax.experimental.pallas import tpu as pltpu
from jax.experimental.pallas import tpu_sc as plsc

_N_WT = 64
_L = 512
_SR = 16000
_B = 8
_T = 64000
_BLOCK = 160
_NBLK = _T // _BLOCK
_ROWS = 500
_LANES = 128

_NW = 32
_WBLK = 13
_WSAMP = _WBLK * _BLOCK


def _prep_body(pitch_ref, wt_ref, att_ref, gmap_ref, gidx_ref, alpha_ref,
               m_ref, d_ref):
    w = wt_ref[...]
    w = jnp.concatenate([w[:4], jnp.tanh(w[4:])], axis=0)
    a = att_ref[...]
    a = a - jnp.max(a, axis=0, keepdims=True)
    e = jnp.exp(a)
    att = e / jnp.sum(e, axis=0, keepdims=True)
    m = lax.dot_general(att, w, (((0,), (0,)), ((), ())),
                        preferred_element_type=jnp.float32)
    m_ref[...] = m
    d_ref[...] = jnp.concatenate([m[:, 1:], m[:, :1]], axis=1) - m

    inc2 = pitch_ref[...] / jnp.float32(_SR) * jnp.float32(_L)
    ui = lax.broadcasted_iota(jnp.int32, (_LANES, _LANES), 0)
    uj = lax.broadcasted_iota(jnp.int32, (_LANES, _LANES), 1)
    u128 = (ui <= uj).astype(jnp.float32)
    y2 = lax.dot_general(inc2, u128, (((1,), (0,)), ((), ())),
                         preferred_element_type=jnp.float32)
    y3 = y2.reshape(_B, _ROWS, _LANES)
    inc3 = inc2.reshape(_B, _ROWS, _LANES)
    rows = y3[:, :, _LANES - 1]
    rows = rows - jnp.float32(_L) * jnp.floor(rows * jnp.float32(1.0 / _L))
    hi = jnp.floor(rows)
    fr = rows - hi
    si = lax.broadcasted_iota(jnp.int32, (_ROWS, _ROWS), 0)
    sj = lax.broadcasted_iota(jnp.int32, (_ROWS, _ROWS), 1)
    su = (si < sj).astype(jnp.float32)
    exhi = lax.dot_general(hi, su, (((1,), (0,)), ((), ())),
                           preferred_element_type=jnp.float32)
    exfr = lax.dot_general(fr, su, (((1,), (0,)), ((), ())),
                           preferred_element_type=jnp.float32)
    exhi = exhi - jnp.float32(_L) * jnp.floor(exhi * jnp.float32(1.0 / _L))
    ex = exhi + exfr
    idx = y3 + ex[:, :, None] - inc3[0:1]
    ph = idx - jnp.float32(_L) * jnp.floor(idx * jnp.float32(1.0 / _L))
    ph = jnp.where(ph >= jnp.float32(_L), ph - jnp.float32(_L), ph)
    lo = jnp.floor(ph)
    alpha_ref[...] = ph - lo
    loi = jnp.minimum(lo.astype(jnp.int32), _L - 1)
    gidx_ref[...] = gmap_ref[...] + loi


def _sc_body(g_hbm, al_hbm, amp_hbm, m_hbm, d_hbm, out_hbm,
             g_v, al_v, amp_v, out_v, mt_v, dt_v, sem):
    cid = lax.axis_index("c")
    sid = lax.axis_index("s")
    wid = sid * 2 + cid
    blk0 = (wid * 25) // 2
    t0 = blk0 * _BLOCK

    cp = pltpu.make_async_copy
    dmas = [
        cp(m_hbm.at[pl.ds(blk0 * _L, _WBLK * _L)], mt_v, sem),
        cp(d_hbm.at[pl.ds(blk0 * _L, _WBLK * _L)], dt_v, sem),
    ]
    for b in range(_B):
        src = pl.ds(b * _T + t0, _WSAMP)
        dst = pl.ds(b * _WSAMP, _WSAMP)
        dmas.append(cp(g_hbm.at[src], g_v.at[dst], sem))
        dmas.append(cp(al_hbm.at[src], al_v.at[dst], sem))
        dmas.append(cp(amp_hbm.at[src], amp_v.at[dst], sem))
    for dma in dmas:
        dma.start()
    for dma in dmas:
        dma.wait()

    gbase = blk0 * _L

    def body(i, carry):
        off = i * 16
        g = g_v[pl.ds(off, 16)] - gbase
        alpha = al_v[pl.ds(off, 16)]
        amp = amp_v[pl.ds(off, 16)]
        mval = plsc.load_gather(mt_v, [g])
        dval = plsc.load_gather(dt_v, [g])
        out_v[pl.ds(off, 16)] = amp * (mval + alpha * dval)
        return carry

    lax.fori_loop(0, _B * _WSAMP // 16, body, 0)

    odmas = [cp(out_v.at[pl.ds(b * _WSAMP, _WSAMP)],
                out_hbm.at[pl.ds(b * _T + t0, _WSAMP)], sem)
             for b in range(_B)]
    for dma in odmas:
        dma.start()
    for dma in odmas:
        dma.wait()


def kernel(pitch, amplitude, wavetables, attention):
    pitch2 = pitch.reshape(_B * _ROWS, _LANES)
    gmap = ((jnp.arange(_T, dtype=jnp.int32) // _BLOCK) * _L).reshape(
        _ROWS, _LANES)[None]
    gidx, alpha, m, d = pl.pallas_call(
        _prep_body,
        out_shape=(
            jax.ShapeDtypeStruct((_B, _ROWS, _LANES), jnp.int32),
            jax.ShapeDtypeStruct((_B, _ROWS, _LANES), jnp.float32),
            jax.ShapeDtypeStruct((_NBLK, _L), jnp.float32),
            jax.ShapeDtypeStruct((_NBLK, _L), jnp.float32),
        ),
    )(pitch2, wavetables, attention, gmap)

    mesh = plsc.VectorSubcoreMesh(core_axis_name="c", subcore_axis_name="s")
    sc = functools.partial(
        pl.kernel,
        mesh=mesh,
        compiler_params=pltpu.CompilerParams(needs_layout_passes=False),
        out_type=jax.ShapeDtypeStruct((_B * _T,), jnp.float32),
        scratch_types=[
            pltpu.VMEM((_B * _WSAMP,), jnp.int32),
            pltpu.VMEM((_B * _WSAMP,), jnp.float32),
            pltpu.VMEM((_B * _WSAMP,), jnp.float32),
            pltpu.VMEM((_B * _WSAMP,), jnp.float32),
            pltpu.VMEM((_WBLK * _L,), jnp.float32),
            pltpu.VMEM((_WBLK * _L,), jnp.float32),
            pltpu.SemaphoreType.DMA,
        ],
    )(_sc_body)
    out = sc(gidx.reshape(_B * _T), alpha.reshape(_B * _T),
             amplitude.reshape(_B * _T), m.reshape(_NBLK * _L),
             d.reshape(_NBLK * _L))
    return out.reshape(_B, _T, 1)
